# baseline (speedup 1.0000x reference)
import functools

import jax
import jax.numpy as jnp
from jax import lax
from jax.experimental import pallas as pl
from jax.experimental.pallas import tpu as pltpu
from jax.experimental.pallas import tpu_sc as plsc

_INFO = plsc.get_sparse_core_info()
_NC = _INFO.num_cores
_NS = _INFO.num_subcores
_NW = _NC * _NS
_L = _INFO.num_lanes


def _body(b_per_w, chunk, d,
          wu_hbm, wi_hbm, wc_hbm, iu_hbm, ii_hbm, ic_hbm, out_hbm,
          iu_v, ii_v, ic_v, iu2_v, ii2_v, ic2_v, u_v, i_v, c_v, r_v,
          sem_u, sem_i, sem_c):
    wid = lax.axis_index("s") * _NC + lax.axis_index("c")
    base = wid * b_per_w

    pltpu.sync_copy(iu_hbm.at[pl.ds(base, b_per_w)], iu_v)
    pltpu.sync_copy(ii_hbm.at[pl.ds(base, b_per_w)], ii_v)
    pltpu.sync_copy(ic_hbm.at[pl.ds(base, b_per_w)], ic_v)

    def halve(g, carry):
        sl = pl.ds(g * _L, _L)
        iu2_v[sl] = iu_v[sl] >> 1
        ii2_v[sl] = ii_v[sl] >> 1
        ic2_v[sl] = ic_v[sl] >> 1
        return carry

    lax.fori_loop(0, b_per_w // _L, halve, 0, unroll=4)

    nch = d // _L

    def step(k, carry):
        ks = k * chunk
        cp_u = pltpu.async_copy(
            wu_hbm.at[iu2_v.at[pl.ds(ks, chunk)]], u_v, sem_u)
        cp_i = pltpu.async_copy(
            wi_hbm.at[ii2_v.at[pl.ds(ks, chunk)]], i_v, sem_i)
        cp_c = pltpu.async_copy(
            wc_hbm.at[ic2_v.at[pl.ds(ks, chunk)]], c_v, sem_c)
        cp_u.wait()
        cp_i.wait()
        cp_c.wait()

        for g in range(chunk // _L):
            uvec = iu_v[pl.ds(ks + g * _L, _L)]
            ivec = ii_v[pl.ds(ks + g * _L, _L)]
            cvec = ic_v[pl.ds(ks + g * _L, _L)]
            for l in range(_L):
                row = g * _L + l
                ou = (uvec[l] & 1) * 64
                oi = (ivec[l] & 1) * 64
                oc = (cvec[l] & 1) * 64
                half = (row & 1) * 64
                for ch in range(nch):
                    r_v[row >> 1, pl.ds(half + ch * _L, _L)] = (
                        u_v[row, pl.ds(ou + ch * _L, _L)]
                        + i_v[row, pl.ds(oi + ch * _L, _L)]
                    ) * c_v[row, pl.ds(oc + ch * _L, _L)]

        obase = pl.multiple_of((base + ks) >> 1, 8)
        pltpu.sync_copy(r_v, out_hbm.at[pl.ds(obase, chunk >> 1)])
        return carry

    lax.fori_loop(0, b_per_w // chunk, step, 0)


def kernel(idx_user_id, idx_item_id, idx_context_id, W_user, W_item, W_ctx):
    b = idx_user_id.shape[0]
    d = W_user.shape[1]
    b_per_w = b // _NW
    chunk = 128
    dd = 2 * d

    mesh = plsc.VectorSubcoreMesh(core_axis_name="c", subcore_axis_name="s")
    f = pl.kernel(
        functools.partial(_body, b_per_w, chunk, d),
        out_type=jax.ShapeDtypeStruct((b // 2, dd), jnp.float32),
        mesh=mesh,
        compiler_params=pltpu.CompilerParams(skip_device_barrier=True),
        scratch_types=[
            pltpu.VMEM((b_per_w,), jnp.int32),
            pltpu.VMEM((b_per_w,), jnp.int32),
            pltpu.VMEM((b_per_w,), jnp.int32),
            pltpu.VMEM((b_per_w,), jnp.int32),
            pltpu.VMEM((b_per_w,), jnp.int32),
            pltpu.VMEM((b_per_w,), jnp.int32),
            pltpu.VMEM((chunk, dd), jnp.float32),
            pltpu.VMEM((chunk, dd), jnp.float32),
            pltpu.VMEM((chunk, dd), jnp.float32),
            pltpu.VMEM((chunk // 2, dd), jnp.float32),
            pltpu.SemaphoreType.DMA,
            pltpu.SemaphoreType.DMA,
            pltpu.SemaphoreType.DMA,
        ],
    )
    out2 = f(W_user.reshape(-1, dd), W_item.reshape(-1, dd),
             W_ctx.reshape(-1, dd),
             idx_user_id, idx_item_id, idx_context_id)
    return out2.reshape(b, d)


# speedup vs baseline: 1.0327x; 1.0327x over previous
import functools

import jax
import jax.numpy as jnp
from jax import lax
from jax.experimental import pallas as pl
from jax.experimental.pallas import tpu as pltpu
from jax.experimental.pallas import tpu_sc as plsc

_INFO = plsc.get_sparse_core_info()
_NC = _INFO.num_cores
_NS = _INFO.num_subcores
_NW = _NC * _NS
_L = _INFO.num_lanes


def _body(b_per_w, d,
          wu_hbm, wi_hbm, wc_hbm, iu_hbm, ii_hbm, ic_hbm, out_hbm,
          iu_v, ii_v, ic_v, u_v, i_v, c_v,
          sem_u, sem_i, sem_c):
    wid = lax.axis_index("s") * _NC + lax.axis_index("c")
    base = wid * b_per_w

    pltpu.sync_copy(iu_hbm.at[pl.ds(base, b_per_w)], iu_v)
    pltpu.sync_copy(ii_hbm.at[pl.ds(base, b_per_w)], ii_v)
    pltpu.sync_copy(ic_hbm.at[pl.ds(base, b_per_w)], ic_v)

    cp_u = pltpu.async_copy(wu_hbm.at[iu_v], u_v, sem_u)
    cp_i = pltpu.async_copy(wi_hbm.at[ii_v], i_v, sem_i)
    cp_c = pltpu.async_copy(wc_hbm.at[ic_v], c_v, sem_c)
    cp_u.wait()
    cp_i.wait()
    cp_c.wait()

    nchunks = d // _L

    def row(r, carry):
        for ch in range(nchunks):
            sl = pl.ds(ch * _L, _L)
            u_v[r, sl] = (u_v[r, sl] + i_v[r, sl]) * c_v[r, sl]
        return carry

    lax.fori_loop(0, b_per_w, row, 0, unroll=4)

    pltpu.sync_copy(u_v, out_hbm.at[pl.ds(base, b_per_w)])


def kernel(idx_user_id, idx_item_id, idx_context_id, W_user, W_item, W_ctx):
    b = idx_user_id.shape[0]
    d = W_user.shape[1]
    b_per_w = b // _NW

    mesh = plsc.VectorSubcoreMesh(core_axis_name="c", subcore_axis_name="s")
    f = pl.kernel(
        functools.partial(_body, b_per_w, d),
        out_type=jax.ShapeDtypeStruct((b, d), jnp.float32),
        mesh=mesh,
        compiler_params=pltpu.CompilerParams(use_tc_tiling_on_sc=False),
        scratch_types=[
            pltpu.VMEM((b_per_w,), jnp.int32),
            pltpu.VMEM((b_per_w,), jnp.int32),
            pltpu.VMEM((b_per_w,), jnp.int32),
            pltpu.VMEM((b_per_w, d), jnp.float32),
            pltpu.VMEM((b_per_w, d), jnp.float32),
            pltpu.VMEM((b_per_w, d), jnp.float32),
            pltpu.SemaphoreType.DMA,
            pltpu.SemaphoreType.DMA,
            pltpu.SemaphoreType.DMA,
        ],
    )
    return f(W_user, W_item, W_ctx, idx_user_id, idx_item_id, idx_context_id)
